# baseline (speedup 1.0000x reference)
import functools
import math

import jax
import jax.numpy as jnp
from jax import lax
from jax.experimental import pallas as pl
from jax.experimental.pallas import tpu as pltpu
from jax.experimental.pallas import tpu_sc as plsc

N = 10000
E = 320000
D = 128
PW = 16
PT = 8
NC, NS = 2, 16
NW = NC * NS
EPW = E // NW
IW = 80
GK = 5
GCH = GK * IW
GNCH = EPW // GCH
SNCH = EPW // IW
STRIDE = 632
N2 = NS * STRIDE

_MESH_KW = dict(core_axis_name="c", subcore_axis_name="s",
                num_cores=NC, num_subcores=NS)


def _sc_gather_body(feat_hbm, snd_hbm, rcv_hbm,
                    fs_out, fr_out,
                    sidx, ridx, fsbuf, frbuf, sem):
    wid = lax.axis_index("s") * NC + lax.axis_index("c")
    base = wid * EPW

    def step(k, carry):
        start = base + k * GCH
        pltpu.sync_copy(snd_hbm.at[pl.ds(start, GCH)], sidx)
        pltpu.sync_copy(rcv_hbm.at[pl.ds(start, GCH)], ridx)
        ds = []
        for j in range(GK):
            sl = pl.ds(j * IW, IW)
            ds.append(pltpu.async_copy(feat_hbm.at[sidx.at[sl]], fsbuf.at[sl], sem))
            ds.append(pltpu.async_copy(feat_hbm.at[ridx.at[sl]], frbuf.at[sl], sem))
        for d in ds:
            d.wait()
        pltpu.sync_copy(fsbuf, fs_out.at[pl.ds(start, GCH)])
        pltpu.sync_copy(frbuf, fr_out.at[pl.ds(start, GCH)])
        return carry

    lax.fori_loop(0, GNCH, step, 0)


@functools.lru_cache(maxsize=None)
def _sc_gather_kernel():
    mesh = plsc.VectorSubcoreMesh(**_MESH_KW)
    return pl.kernel(
        _sc_gather_body,
        out_type=[
            jax.ShapeDtypeStruct((E, D), jnp.float32),
            jax.ShapeDtypeStruct((E, D), jnp.float32),
        ],
        mesh=mesh,
        scratch_types=[
            pltpu.VMEM((GCH,), jnp.int32),
            pltpu.VMEM((GCH,), jnp.int32),
            pltpu.VMEM((GCH, D), jnp.float32),
            pltpu.VMEM((GCH, D), jnp.float32),
            pltpu.SemaphoreType.DMA,
        ],
        compiler_params=pltpu.CompilerParams(needs_layout_passes=False),
    )


def _sc_geom_body(pos_hbm, snd_hbm, rcv_hbm, geo_out,
                  postab, sidx, ridx, geobuf):
    wid = lax.axis_index("s") * NC + lax.axis_index("c")
    base = wid * EPW
    pltpu.sync_copy(pos_hbm, postab)
    lane = lax.iota(jnp.int32, 16)
    zero = jnp.zeros((16,), jnp.float32)

    def chunk(k, carry):
        start = base + k * GCH
        pltpu.sync_copy(snd_hbm.at[pl.ds(start, GCH)], sidx)
        pltpu.sync_copy(rcv_hbm.at[pl.ds(start, GCH)], ridx)

        def group(g, carry2):
            ids_s = sidx[pl.ds(g * 16, 16)] * PT
            ids_r = ridx[pl.ds(g * 16, 16)] * PT
            rowb = g * (16 * PW) + lane * PW
            v = []
            for c in range(6):
                pcs = plsc.load_gather(postab, [ids_s + c])
                pcr = plsc.load_gather(postab, [ids_r + c])
                vc = pcr - pcs
                plsc.store_scatter(geobuf, [rowb + c], vc)
                v.append(vc)
            sq0 = v[0] * v[0] + v[1] * v[1] + v[2] * v[2]
            sq1 = v[3] * v[3] + v[4] * v[4] + v[5] * v[5]
            plsc.store_scatter(geobuf, [rowb + 6], sq0)
            plsc.store_scatter(geobuf, [rowb + 7], sq1)
            for c in range(8, PW):
                plsc.store_scatter(geobuf, [rowb + c], zero)
            return carry2

        lax.fori_loop(0, GCH // 16, group, 0)
        pltpu.sync_copy(geobuf, geo_out.at[pl.ds(start * PW, GCH * PW)])
        return carry

    lax.fori_loop(0, GNCH, chunk, 0)


@functools.lru_cache(maxsize=None)
def _sc_geom_kernel():
    mesh = plsc.VectorSubcoreMesh(**_MESH_KW)
    return pl.kernel(
        _sc_geom_body,
        out_type=jax.ShapeDtypeStruct((E * PW,), jnp.float32),
        mesh=mesh,
        scratch_types=[
            pltpu.VMEM((N * PT,), jnp.float32),
            pltpu.VMEM((GCH,), jnp.int32),
            pltpu.VMEM((GCH,), jnp.int32),
            pltpu.VMEM((GCH * PW,), jnp.float32),
        ],
        compiler_params=pltpu.CompilerParams(needs_layout_passes=False),
    )


def _sc_scatter_body(src_hbm, rcv_hbm, zero_hbm,
                     out, acc, idx, buf):
    c = lax.axis_index("c")
    s = lax.axis_index("s")
    wid = s * NC + c
    r0 = s * STRIDE
    pltpu.sync_copy(zero_hbm.at[pl.ds(r0, STRIDE)], acc.at[pl.ds(r0, STRIDE)])
    plsc.subcore_barrier()
    base = wid * EPW

    def step(k, carry):
        start = base + k * IW
        pltpu.sync_copy(rcv_hbm.at[pl.ds(start, IW)], idx)
        pltpu.sync_copy(src_hbm.at[pl.ds(start, IW)], buf)
        pltpu.sync_copy(buf, acc.at[idx], add=True)
        return carry

    lax.fori_loop(0, SNCH, step, 0)
    plsc.subcore_barrier()
    pltpu.sync_copy(acc.at[pl.ds(r0, STRIDE)], out.at[c].at[pl.ds(r0, STRIDE)])


@functools.lru_cache(maxsize=None)
def _sc_scatter_kernel():
    mesh = plsc.VectorSubcoreMesh(**_MESH_KW)
    return pl.kernel(
        _sc_scatter_body,
        out_type=jax.ShapeDtypeStruct((NC, N2, D), jnp.float32),
        mesh=mesh,
        scratch_types=[
            pltpu.VMEM_SHARED((N2, D), jnp.float32),
            pltpu.VMEM((IW,), jnp.int32),
            pltpu.VMEM((IW, D), jnp.float32),
        ],
        compiler_params=pltpu.CompilerParams(needs_layout_passes=False),
    )


def _dot(a, b):
    return lax.dot_general(a, b, (((1,), (0,)), ((), ())),
                           preferred_element_type=jnp.float32)


def _bdot(a, b):
    return lax.dot_general(a.astype(jnp.bfloat16), b.astype(jnp.bfloat16),
                           (((1,), (0,)), ((), ())),
                           preferred_element_type=jnp.float32)


def _edge_body(fs, fr, geo, w1s, w1r, w1g, b1, w2, b2,
               wt1, bt1, wt2, bt2, wx, bx, wi, bi, m16,
               omsg, oshf):
    g16 = geo[...]
    h = jax.nn.silu(_bdot(fs[...], w1s[...]) + _bdot(fr[...], w1r[...])
                    + _dot(g16, w1g[...]) + b1[...])
    m = jax.nn.silu(_bdot(h, w2[...]) + b2[...])
    t = jax.nn.silu(_bdot(m, wt1[...]) + bt1[...])
    t = jax.nn.silu(_bdot(t, wt2[...]) + bt2[...])
    px = _dot(t, wx[...]) + bx[...]
    sq16 = _dot(g16, m16[...])
    ln = jnp.sqrt(jnp.where(sq16 == 0.0, 1e-20, sq16))
    e = jax.nn.sigmoid(_dot(m, wi[...]) + bi[...])
    omsg[...] = m * e[:, 0:1]
    shift16 = px * g16 / (1.0 + ln)
    oshf[...] = jnp.concatenate(
        [shift16, jnp.zeros((shift16.shape[0], D - PW), jnp.float32)], axis=1)


def _node_body(am, ash, feat, pos, wh1m, wh1f, bh1, wh2, bh2, wh3, bh3,
               ofeat, opos):
    amv = am[...]
    asv = ash[...]
    m_i = (amv[0] + amv[1]) * (1.0 / math.sqrt(float(N - 1)))
    s_i = (asv[0] + asv[1])[:, :PW] * (1.0 / float(N - 1))
    f = feat[...]
    h = jax.nn.silu(_bdot(m_i, wh1m[...]) + _bdot(f, wh1f[...]) + bh1[...])
    h = jax.nn.silu(_bdot(h, wh2[...]) + bh2[...])
    ofeat[...] = _bdot(h, wh3[...]) + bh3[...] + f
    opos[...] = pos[...] + s_i


BE = 2560
BN = 1000


def _full(shape):
    return pl.BlockSpec(shape, lambda i: (0,) * len(shape))


def _edge_mlp(fs, fr, geo, ws):
    specs = [
        pl.BlockSpec((BE, D), lambda i: (i, 0)),
        pl.BlockSpec((BE, D), lambda i: (i, 0)),
        pl.BlockSpec((BE, PW), lambda i: (i, 0)),
    ] + [_full(w.shape) for w in ws]
    return pl.pallas_call(
        _edge_body,
        grid=(E // BE,),
        in_specs=specs,
        out_specs=[
            pl.BlockSpec((BE, D), lambda i: (i, 0)),
            pl.BlockSpec((BE, D), lambda i: (i, 0)),
        ],
        out_shape=[
            jax.ShapeDtypeStruct((E, D), jnp.float32),
            jax.ShapeDtypeStruct((E, D), jnp.float32),
        ],
    )(fs, fr, geo, *ws)


def _node_mlp(am, ash, feat, pos, ws):
    specs = [
        pl.BlockSpec((NC, BN, D), lambda i: (0, i, 0)),
        pl.BlockSpec((NC, BN, D), lambda i: (0, i, 0)),
        pl.BlockSpec((BN, D), lambda i: (i, 0)),
        pl.BlockSpec((BN, PW), lambda i: (i, 0)),
    ] + [_full(w.shape) for w in ws]
    return pl.pallas_call(
        _node_body,
        grid=(N // BN,),
        in_specs=specs,
        out_specs=[
            pl.BlockSpec((BN, D), lambda i: (i, 0)),
            pl.BlockSpec((BN, PW), lambda i: (i, 0)),
        ],
        out_shape=[
            jax.ShapeDtypeStruct((N, D), jnp.float32),
            jax.ShapeDtypeStruct((N, PW), jnp.float32),
        ],
    )(am, ash, feat, pos, *ws)


def _prep_weights(params):
    f32 = jnp.float32
    (w1, b1), (w2, b2) = params["phi_e"]
    w1s, w1r, w1l = w1[:D], w1[D:2 * D], w1[2 * D:]
    g = jnp.zeros((PW, 2), f32)
    g = g.at[0:3, 0].set(1.0).at[3:6, 1].set(1.0)
    g2 = jnp.zeros((PW, 2), f32)
    g2 = g2.at[6, 0].set(1.0).at[7, 1].set(1.0)
    w1g = g2 @ w1l
    m16 = g2 @ g.T
    (wt1, bt1), (wt2, bt2) = params["phi_x_torso"]
    wx, bx = params["phi_x_out"]
    wx16 = wx @ g.T
    bx16 = (bx @ g.T)[None, :]
    wi, bi = params["phi_inf"]
    wi8 = jnp.pad(wi, ((0, 0), (0, 7)))
    bi8 = jnp.pad(bi, (0, 7))[None, :]
    (wh1, bh1), (wh2, bh2), (wh3, bh3) = params["phi_h"]
    edge_ws = [w1s, w1r, w1g, b1[None, :], w2, b2[None, :],
               wt1, bt1[None, :], wt2, bt2[None, :],
               wx16, bx16, wi8, bi8, m16]
    node_ws = [wh1[:D], wh1[D:], bh1[None, :], wh2, bh2[None, :],
               wh3, bh3[None, :]]
    return edge_ws, node_ws


def kernel(node_positions, node_features, senders, receivers, params):
    n, v, dim = node_positions.shape
    pos_flat = jnp.pad(node_positions.reshape(n, v * dim).astype(jnp.float32),
                       ((0, 0), (0, PT - v * dim))).reshape(n * PT)
    snd = senders.astype(jnp.int32)
    rcv = receivers.astype(jnp.int32)
    edge_ws, node_ws = _prep_weights(params)

    fs, fr = _sc_gather_kernel()(node_features, snd, rcv)
    geo = _sc_geom_kernel()(pos_flat, snd, rcv).reshape(E, PW)
    msg, shf = _edge_mlp(fs, fr, geo, edge_ws)
    zeros = jnp.zeros((N2, D), jnp.float32)
    am = _sc_scatter_kernel()(msg, rcv, zeros)
    ash = _sc_scatter_kernel()(shf, rcv, zeros)
    pos16 = jnp.pad(node_positions.reshape(n, v * dim).astype(jnp.float32),
                    ((0, 0), (0, PW - v * dim)))
    feats_out, pos_out = _node_mlp(am, ash, node_features, pos16, node_ws)
    vectors_out = pos_out[:, :v * dim].reshape(n, v, dim)
    return (vectors_out, feats_out)


# speedup vs baseline: 1.0311x; 1.0311x over previous
import functools
import math

import jax
import jax.numpy as jnp
from jax import lax
from jax.experimental import pallas as pl
from jax.experimental.pallas import tpu as pltpu
from jax.experimental.pallas import tpu_sc as plsc

N = 10000
E = 320000
D = 128
PW = 16
PT = 8
NC, NS = 2, 16
NW = NC * NS
EPW = E // NW
IW = 80
GK = 5
GCH = GK * IW
GNCH = EPW // GCH
SNCH = EPW // IW
STRIDE = 632
N2 = NS * STRIDE

_MESH_KW = dict(core_axis_name="c", subcore_axis_name="s",
                num_cores=NC, num_subcores=NS)


def _sc_gather_body(feat_hbm, snd_hbm, rcv_hbm,
                    fs_out, fr_out,
                    sidx, ridx, fsbuf, frbuf, sem):
    wid = lax.axis_index("s") * NC + lax.axis_index("c")
    base = wid * EPW

    def step(k, carry):
        start = base + k * GCH
        pltpu.sync_copy(snd_hbm.at[pl.ds(start, GCH)], sidx)
        pltpu.sync_copy(rcv_hbm.at[pl.ds(start, GCH)], ridx)
        ds = []
        for j in range(GK):
            sl = pl.ds(j * IW, IW)
            ds.append(pltpu.async_copy(feat_hbm.at[sidx.at[sl]], fsbuf.at[sl], sem))
            ds.append(pltpu.async_copy(feat_hbm.at[ridx.at[sl]], frbuf.at[sl], sem))
        for d in ds:
            d.wait()
        pltpu.sync_copy(fsbuf, fs_out.at[pl.ds(start, GCH)])
        pltpu.sync_copy(frbuf, fr_out.at[pl.ds(start, GCH)])
        return carry

    lax.fori_loop(0, GNCH, step, 0)


@functools.lru_cache(maxsize=None)
def _sc_gather_kernel():
    mesh = plsc.VectorSubcoreMesh(**_MESH_KW)
    return pl.kernel(
        _sc_gather_body,
        out_type=[
            jax.ShapeDtypeStruct((E, D), jnp.float32),
            jax.ShapeDtypeStruct((E, D), jnp.float32),
        ],
        mesh=mesh,
        scratch_types=[
            pltpu.VMEM((GCH,), jnp.int32),
            pltpu.VMEM((GCH,), jnp.int32),
            pltpu.VMEM((GCH, D), jnp.float32),
            pltpu.VMEM((GCH, D), jnp.float32),
            pltpu.SemaphoreType.DMA,
        ],
        compiler_params=pltpu.CompilerParams(needs_layout_passes=False),
    )


def _sc_geom_body(pos_hbm, snd_hbm, rcv_hbm, geo_out,
                  postab, sidx, ridx, geobuf):
    wid = lax.axis_index("s") * NC + lax.axis_index("c")
    base = wid * EPW
    pltpu.sync_copy(pos_hbm, postab)
    lane = lax.iota(jnp.int32, 16)
    zero = jnp.zeros((16,), jnp.float32)

    def chunk(k, carry):
        start = base + k * GCH
        pltpu.sync_copy(snd_hbm.at[pl.ds(start, GCH)], sidx)
        pltpu.sync_copy(rcv_hbm.at[pl.ds(start, GCH)], ridx)

        def group(g, carry2):
            ids_s = sidx[pl.ds(g * 16, 16)] * PT
            ids_r = ridx[pl.ds(g * 16, 16)] * PT
            rowb = g * (16 * PW) + lane * PW
            v = []
            for c in range(6):
                pcs = plsc.load_gather(postab, [ids_s + c])
                pcr = plsc.load_gather(postab, [ids_r + c])
                vc = pcr - pcs
                plsc.store_scatter(geobuf, [rowb + c], vc)
                v.append(vc)
            sq0 = v[0] * v[0] + v[1] * v[1] + v[2] * v[2]
            sq1 = v[3] * v[3] + v[4] * v[4] + v[5] * v[5]
            plsc.store_scatter(geobuf, [rowb + 6], sq0)
            plsc.store_scatter(geobuf, [rowb + 7], sq1)
            for c in range(8, PW):
                plsc.store_scatter(geobuf, [rowb + c], zero)
            return carry2

        lax.fori_loop(0, GCH // 16, group, 0)
        pltpu.sync_copy(geobuf, geo_out.at[pl.ds(start * PW, GCH * PW)])
        return carry

    lax.fori_loop(0, GNCH, chunk, 0)


@functools.lru_cache(maxsize=None)
def _sc_geom_kernel():
    mesh = plsc.VectorSubcoreMesh(**_MESH_KW)
    return pl.kernel(
        _sc_geom_body,
        out_type=jax.ShapeDtypeStruct((E * PW,), jnp.float32),
        mesh=mesh,
        scratch_types=[
            pltpu.VMEM((N * PT,), jnp.float32),
            pltpu.VMEM((GCH,), jnp.int32),
            pltpu.VMEM((GCH,), jnp.int32),
            pltpu.VMEM((GCH * PW,), jnp.float32),
        ],
        compiler_params=pltpu.CompilerParams(needs_layout_passes=False),
    )


def _sc_scatter_body(src_hbm, rcv_hbm, zero_hbm,
                     out, acc, idx, buf):
    c = lax.axis_index("c")
    s = lax.axis_index("s")
    wid = s * NC + c
    r0 = s * STRIDE
    pltpu.sync_copy(zero_hbm.at[pl.ds(r0, STRIDE)], acc.at[pl.ds(r0, STRIDE)])
    plsc.subcore_barrier()
    base = wid * EPW

    def step(k, carry):
        start = base + k * IW
        pltpu.sync_copy(rcv_hbm.at[pl.ds(start, IW)], idx)
        pltpu.sync_copy(src_hbm.at[pl.ds(start, IW)], buf)
        pltpu.sync_copy(buf, acc.at[idx], add=True)
        return carry

    lax.fori_loop(0, SNCH, step, 0)
    plsc.subcore_barrier()
    pltpu.sync_copy(acc.at[pl.ds(r0, STRIDE)], out.at[c].at[pl.ds(r0, STRIDE)])


@functools.lru_cache(maxsize=None)
def _sc_scatter_kernel(width=D):
    mesh = plsc.VectorSubcoreMesh(**_MESH_KW)
    return pl.kernel(
        _sc_scatter_body,
        out_type=jax.ShapeDtypeStruct((NC, N2, width), jnp.float32),
        mesh=mesh,
        scratch_types=[
            pltpu.VMEM_SHARED((N2, width), jnp.float32),
            pltpu.VMEM((IW,), jnp.int32),
            pltpu.VMEM((IW, width), jnp.float32),
        ],
        compiler_params=pltpu.CompilerParams(needs_layout_passes=False),
    )


def _dot(a, b):
    return lax.dot_general(a, b, (((1,), (0,)), ((), ())),
                           preferred_element_type=jnp.float32)


def _bdot(a, b):
    return lax.dot_general(a.astype(jnp.bfloat16), b.astype(jnp.bfloat16),
                           (((1,), (0,)), ((), ())),
                           preferred_element_type=jnp.float32)


def _edge_body(fs, fr, geo, w1s, w1r, w1g, b1, w2, b2,
               wt1, bt1, wt2, bt2, wx, bx, wi, bi, m16,
               omsg, oshf):
    g16 = geo[...]
    h = jax.nn.silu(_bdot(fs[...], w1s[...]) + _bdot(fr[...], w1r[...])
                    + _dot(g16, w1g[...]) + b1[...])
    m = jax.nn.silu(_bdot(h, w2[...]) + b2[...])
    t = jax.nn.silu(_bdot(m, wt1[...]) + bt1[...])
    t = jax.nn.silu(_bdot(t, wt2[...]) + bt2[...])
    px = _dot(t, wx[...]) + bx[...]
    sq16 = _dot(g16, m16[...])
    ln = jnp.sqrt(jnp.where(sq16 == 0.0, 1e-20, sq16))
    e = jax.nn.sigmoid(_dot(m, wi[...]) + bi[...])
    omsg[...] = m * e[:, 0:1]
    oshf[...] = px * g16 / (1.0 + ln)


def _node_body(am, ash, feat, pos, wh1m, wh1f, bh1, wh2, bh2, wh3, bh3,
               ofeat, opos):
    amv = am[...]
    asv = ash[...]
    m_i = (amv[0] + amv[1]) * (1.0 / math.sqrt(float(N - 1)))
    s_i = (asv[0] + asv[1]) * (1.0 / float(N - 1))
    f = feat[...]
    h = jax.nn.silu(_bdot(m_i, wh1m[...]) + _bdot(f, wh1f[...]) + bh1[...])
    h = jax.nn.silu(_bdot(h, wh2[...]) + bh2[...])
    ofeat[...] = _bdot(h, wh3[...]) + bh3[...] + f
    opos[...] = pos[...] + s_i


BE = 2560
BN = 1000


def _full(shape):
    return pl.BlockSpec(shape, lambda i: (0,) * len(shape))


def _edge_mlp(fs, fr, geo, ws):
    specs = [
        pl.BlockSpec((BE, D), lambda i: (i, 0)),
        pl.BlockSpec((BE, D), lambda i: (i, 0)),
        pl.BlockSpec((BE, PW), lambda i: (i, 0)),
    ] + [_full(w.shape) for w in ws]
    return pl.pallas_call(
        _edge_body,
        grid=(E // BE,),
        in_specs=specs,
        out_specs=[
            pl.BlockSpec((BE, D), lambda i: (i, 0)),
            pl.BlockSpec((BE, PW), lambda i: (i, 0)),
        ],
        out_shape=[
            jax.ShapeDtypeStruct((E, D), jnp.float32),
            jax.ShapeDtypeStruct((E, PW), jnp.float32),
        ],
    )(fs, fr, geo, *ws)


def _node_mlp(am, ash, feat, pos, ws):
    specs = [
        pl.BlockSpec((NC, BN, D), lambda i: (0, i, 0)),
        pl.BlockSpec((NC, BN, PW), lambda i: (0, i, 0)),
        pl.BlockSpec((BN, D), lambda i: (i, 0)),
        pl.BlockSpec((BN, PW), lambda i: (i, 0)),
    ] + [_full(w.shape) for w in ws]
    return pl.pallas_call(
        _node_body,
        grid=(N // BN,),
        in_specs=specs,
        out_specs=[
            pl.BlockSpec((BN, D), lambda i: (i, 0)),
            pl.BlockSpec((BN, PW), lambda i: (i, 0)),
        ],
        out_shape=[
            jax.ShapeDtypeStruct((N, D), jnp.float32),
            jax.ShapeDtypeStruct((N, PW), jnp.float32),
        ],
    )(am, ash, feat, pos, *ws)


def _prep_weights(params):
    f32 = jnp.float32
    (w1, b1), (w2, b2) = params["phi_e"]
    w1s, w1r, w1l = w1[:D], w1[D:2 * D], w1[2 * D:]
    g = jnp.zeros((PW, 2), f32)
    g = g.at[0:3, 0].set(1.0).at[3:6, 1].set(1.0)
    g2 = jnp.zeros((PW, 2), f32)
    g2 = g2.at[6, 0].set(1.0).at[7, 1].set(1.0)
    w1g = g2 @ w1l
    m16 = g2 @ g.T
    (wt1, bt1), (wt2, bt2) = params["phi_x_torso"]
    wx, bx = params["phi_x_out"]
    wx16 = wx @ g.T
    bx16 = (bx @ g.T)[None, :]
    wi, bi = params["phi_inf"]
    wi8 = jnp.pad(wi, ((0, 0), (0, 7)))
    bi8 = jnp.pad(bi, (0, 7))[None, :]
    (wh1, bh1), (wh2, bh2), (wh3, bh3) = params["phi_h"]
    edge_ws = [w1s, w1r, w1g, b1[None, :], w2, b2[None, :],
               wt1, bt1[None, :], wt2, bt2[None, :],
               wx16, bx16, wi8, bi8, m16]
    node_ws = [wh1[:D], wh1[D:], bh1[None, :], wh2, bh2[None, :],
               wh3, bh3[None, :]]
    return edge_ws, node_ws


def kernel(node_positions, node_features, senders, receivers, params):
    n, v, dim = node_positions.shape
    pos_flat = jnp.pad(node_positions.reshape(n, v * dim).astype(jnp.float32),
                       ((0, 0), (0, PT - v * dim))).reshape(n * PT)
    snd = senders.astype(jnp.int32)
    rcv = receivers.astype(jnp.int32)
    edge_ws, node_ws = _prep_weights(params)

    fs, fr = _sc_gather_kernel()(node_features, snd, rcv)
    geo = _sc_geom_kernel()(pos_flat, snd, rcv).reshape(E, PW)
    msg, shf = _edge_mlp(fs, fr, geo, edge_ws)
    zeros = jnp.zeros((N2, D), jnp.float32)
    am = _sc_scatter_kernel(D)(msg, rcv, zeros)
    ash = _sc_scatter_kernel(PW)(shf, rcv, jnp.zeros((N2, PW), jnp.float32))
    pos16 = jnp.pad(node_positions.reshape(n, v * dim).astype(jnp.float32),
                    ((0, 0), (0, PW - v * dim)))
    feats_out, pos_out = _node_mlp(am, ash, node_features, pos16, node_ws)
    vectors_out = pos_out[:, :v * dim].reshape(n, v, dim)
    return (vectors_out, feats_out)


# speedup vs baseline: 1.2012x; 1.1650x over previous
import functools
import math

import jax
import jax.numpy as jnp
from jax import lax
from jax.experimental import pallas as pl
from jax.experimental.pallas import tpu as pltpu
from jax.experimental.pallas import tpu_sc as plsc

N = 10000
E = 320000
D = 128
PW = 16
PT = 8
NC, NS = 2, 16
NW = NC * NS
EPW = E // NW
IW = 80
GK = 5
GCH = GK * IW
GNCH = EPW // GCH
SNCH = EPW // IW
STRIDE = 632
N2 = NS * STRIDE

_MESH_KW = dict(core_axis_name="c", subcore_axis_name="s",
                num_cores=NC, num_subcores=NS)


def _sc_gather_body(feat_hbm, snd_hbm, rcv_hbm,
                    fs_out, fr_out,
                    sidx, ridx, fsbuf, frbuf, sem):
    wid = lax.axis_index("s") * NC + lax.axis_index("c")
    base = wid * EPW

    def step(k, carry):
        start = base + k * GCH
        pltpu.sync_copy(snd_hbm.at[pl.ds(start, GCH)], sidx)
        pltpu.sync_copy(rcv_hbm.at[pl.ds(start, GCH)], ridx)
        ds = []
        for j in range(GK):
            sl = pl.ds(j * IW, IW)
            ds.append(pltpu.async_copy(feat_hbm.at[sidx.at[sl]], fsbuf.at[sl], sem))
            ds.append(pltpu.async_copy(feat_hbm.at[ridx.at[sl]], frbuf.at[sl], sem))
        for d in ds:
            d.wait()
        pltpu.sync_copy(fsbuf, fs_out.at[pl.ds(start, GCH)])
        pltpu.sync_copy(frbuf, fr_out.at[pl.ds(start, GCH)])
        return carry

    lax.fori_loop(0, GNCH, step, 0)


@functools.lru_cache(maxsize=None)
def _sc_gather_kernel():
    mesh = plsc.VectorSubcoreMesh(**_MESH_KW)
    return pl.kernel(
        _sc_gather_body,
        out_type=[
            jax.ShapeDtypeStruct((E, D), jnp.float32),
            jax.ShapeDtypeStruct((E, D), jnp.float32),
        ],
        mesh=mesh,
        scratch_types=[
            pltpu.VMEM((GCH,), jnp.int32),
            pltpu.VMEM((GCH,), jnp.int32),
            pltpu.VMEM((GCH, D), jnp.float32),
            pltpu.VMEM((GCH, D), jnp.float32),
            pltpu.SemaphoreType.DMA,
        ],
        compiler_params=pltpu.CompilerParams(needs_layout_passes=False),
    )


def _sc_geom_body(pos_hbm, snd_hbm, rcv_hbm, geo_out,
                  postab, sidx, ridx, geobuf):
    wid = lax.axis_index("s") * NC + lax.axis_index("c")
    base = wid * EPW
    pltpu.sync_copy(pos_hbm, postab)
    lane = lax.iota(jnp.int32, 16)
    zero = jnp.zeros((16,), jnp.float32)

    def chunk(k, carry):
        start = base + k * GCH
        pltpu.sync_copy(snd_hbm.at[pl.ds(start, GCH)], sidx)
        pltpu.sync_copy(rcv_hbm.at[pl.ds(start, GCH)], ridx)

        def group(g, carry2):
            ids_s = sidx[pl.ds(g * 16, 16)] * PT
            ids_r = ridx[pl.ds(g * 16, 16)] * PT
            rowb = g * (16 * PW) + lane * PW
            v = []
            for c in range(6):
                pcs = plsc.load_gather(postab, [ids_s + c])
                pcr = plsc.load_gather(postab, [ids_r + c])
                vc = pcr - pcs
                plsc.store_scatter(geobuf, [rowb + c], vc)
                v.append(vc)
            sq0 = v[0] * v[0] + v[1] * v[1] + v[2] * v[2]
            sq1 = v[3] * v[3] + v[4] * v[4] + v[5] * v[5]
            plsc.store_scatter(geobuf, [rowb + 6], sq0)
            plsc.store_scatter(geobuf, [rowb + 7], sq1)
            for c in range(8, PW):
                plsc.store_scatter(geobuf, [rowb + c], zero)
            return carry2

        lax.fori_loop(0, GCH // 16, group, 0)
        pltpu.sync_copy(geobuf, geo_out.at[pl.ds(start * PW, GCH * PW)])
        return carry

    lax.fori_loop(0, GNCH, chunk, 0)


@functools.lru_cache(maxsize=None)
def _sc_geom_kernel():
    mesh = plsc.VectorSubcoreMesh(**_MESH_KW)
    return pl.kernel(
        _sc_geom_body,
        out_type=jax.ShapeDtypeStruct((E * PW,), jnp.float32),
        mesh=mesh,
        scratch_types=[
            pltpu.VMEM((N * PT,), jnp.float32),
            pltpu.VMEM((GCH,), jnp.int32),
            pltpu.VMEM((GCH,), jnp.int32),
            pltpu.VMEM((GCH * PW,), jnp.float32),
        ],
        compiler_params=pltpu.CompilerParams(needs_layout_passes=False),
    )


SS = 320
SROWS = SS // IW
NSS = EPW // SS


def _sc_scatter_body(src_hbm, rcv_hbm, zero_hbm,
                     out, acc, idx2, buf, sem):
    c = lax.axis_index("c")
    s = lax.axis_index("s")
    wid = s * NC + c
    r0 = s * STRIDE
    pltpu.sync_copy(zero_hbm.at[pl.ds(r0, STRIDE)], acc.at[pl.ds(r0, STRIDE)])
    plsc.subcore_barrier()
    base = wid * EPW
    row_base = wid * (EPW // IW)

    def round_(k, nrows, start, row):
        a = jnp.minimum((row // 8) * 8, (E // IW) - 16)
        off = row - a
        pltpu.sync_copy(rcv_hbm.at[pl.ds(a, 16)], idx2)
        pltpu.sync_copy(src_hbm.at[pl.ds(start, nrows * IW)],
                        buf.at[pl.ds(0, nrows * IW)])
        ds = []
        for i in range(nrows):
            ds.append(pltpu.async_copy(
                buf.at[pl.ds(i * IW, IW)], acc.at[idx2.at[off + i]], sem,
                add=True))
        for d in ds:
            d.wait()

    def step(k, carry):
        round_(k, SROWS, base + k * SS, row_base + k * SROWS)
        return carry

    lax.fori_loop(0, NSS, step, 0)
    round_(NSS, 1, base + NSS * SS, row_base + NSS * SROWS)
    plsc.subcore_barrier()
    pltpu.sync_copy(acc.at[pl.ds(r0, STRIDE)], out.at[c].at[pl.ds(r0, STRIDE)])


@functools.lru_cache(maxsize=None)
def _sc_scatter_kernel(width=D):
    mesh = plsc.VectorSubcoreMesh(**_MESH_KW)
    return pl.kernel(
        _sc_scatter_body,
        out_type=jax.ShapeDtypeStruct((NC, N2, width), jnp.float32),
        mesh=mesh,
        scratch_types=[
            pltpu.VMEM_SHARED((N2, width), jnp.float32),
            pltpu.VMEM((16, IW), jnp.int32),
            pltpu.VMEM((SS, width), jnp.float32),
            pltpu.SemaphoreType.DMA,
        ],
        compiler_params=pltpu.CompilerParams(needs_layout_passes=False),
    )


def _dot(a, b):
    return lax.dot_general(a, b, (((1,), (0,)), ((), ())),
                           preferred_element_type=jnp.float32)


def _bdot(a, b):
    return lax.dot_general(a.astype(jnp.bfloat16), b.astype(jnp.bfloat16),
                           (((1,), (0,)), ((), ())),
                           preferred_element_type=jnp.float32)


def _edge_body(fs, fr, geo, w1s, w1r, w1g, b1, w2, b2,
               wt1, bt1, wt2, bt2, wx, bx, wi, bi, m16,
               omsg, oshf):
    g16 = geo[...]
    h = jax.nn.silu(_bdot(fs[...], w1s[...]) + _bdot(fr[...], w1r[...])
                    + _dot(g16, w1g[...]) + b1[...])
    m = jax.nn.silu(_bdot(h, w2[...]) + b2[...])
    t = jax.nn.silu(_bdot(m, wt1[...]) + bt1[...])
    t = jax.nn.silu(_bdot(t, wt2[...]) + bt2[...])
    px = _dot(t, wx[...]) + bx[...]
    sq16 = _dot(g16, m16[...])
    ln = jnp.sqrt(jnp.where(sq16 == 0.0, 1e-20, sq16))
    e = jax.nn.sigmoid(_dot(m, wi[...]) + bi[...])
    omsg[...] = m * e[:, 0:1]
    oshf[...] = px * g16 / (1.0 + ln)


def _node_body(am, ash, feat, pos, wh1m, wh1f, bh1, wh2, bh2, wh3, bh3,
               ofeat, opos):
    amv = am[...]
    asv = ash[...]
    m_i = (amv[0] + amv[1]) * (1.0 / math.sqrt(float(N - 1)))
    s_i = (asv[0] + asv[1]) * (1.0 / float(N - 1))
    f = feat[...]
    h = jax.nn.silu(_bdot(m_i, wh1m[...]) + _bdot(f, wh1f[...]) + bh1[...])
    h = jax.nn.silu(_bdot(h, wh2[...]) + bh2[...])
    ofeat[...] = _bdot(h, wh3[...]) + bh3[...] + f
    opos[...] = pos[...] + s_i


BE = 2560
BN = 1000


def _full(shape):
    return pl.BlockSpec(shape, lambda i: (0,) * len(shape))


def _edge_mlp(fs, fr, geo, ws):
    specs = [
        pl.BlockSpec((BE, D), lambda i: (i, 0)),
        pl.BlockSpec((BE, D), lambda i: (i, 0)),
        pl.BlockSpec((BE, PW), lambda i: (i, 0)),
    ] + [_full(w.shape) for w in ws]
    return pl.pallas_call(
        _edge_body,
        grid=(E // BE,),
        in_specs=specs,
        out_specs=[
            pl.BlockSpec((BE, D), lambda i: (i, 0)),
            pl.BlockSpec((BE, PW), lambda i: (i, 0)),
        ],
        out_shape=[
            jax.ShapeDtypeStruct((E, D), jnp.float32),
            jax.ShapeDtypeStruct((E, PW), jnp.float32),
        ],
    )(fs, fr, geo, *ws)


def _node_mlp(am, ash, feat, pos, ws):
    specs = [
        pl.BlockSpec((NC, BN, D), lambda i: (0, i, 0)),
        pl.BlockSpec((NC, BN, PW), lambda i: (0, i, 0)),
        pl.BlockSpec((BN, D), lambda i: (i, 0)),
        pl.BlockSpec((BN, PW), lambda i: (i, 0)),
    ] + [_full(w.shape) for w in ws]
    return pl.pallas_call(
        _node_body,
        grid=(N // BN,),
        in_specs=specs,
        out_specs=[
            pl.BlockSpec((BN, D), lambda i: (i, 0)),
            pl.BlockSpec((BN, PW), lambda i: (i, 0)),
        ],
        out_shape=[
            jax.ShapeDtypeStruct((N, D), jnp.float32),
            jax.ShapeDtypeStruct((N, PW), jnp.float32),
        ],
    )(am, ash, feat, pos, *ws)


def _prep_weights(params):
    f32 = jnp.float32
    (w1, b1), (w2, b2) = params["phi_e"]
    w1s, w1r, w1l = w1[:D], w1[D:2 * D], w1[2 * D:]
    g = jnp.zeros((PW, 2), f32)
    g = g.at[0:3, 0].set(1.0).at[3:6, 1].set(1.0)
    g2 = jnp.zeros((PW, 2), f32)
    g2 = g2.at[6, 0].set(1.0).at[7, 1].set(1.0)
    w1g = g2 @ w1l
    m16 = g2 @ g.T
    (wt1, bt1), (wt2, bt2) = params["phi_x_torso"]
    wx, bx = params["phi_x_out"]
    wx16 = wx @ g.T
    bx16 = (bx @ g.T)[None, :]
    wi, bi = params["phi_inf"]
    wi8 = jnp.pad(wi, ((0, 0), (0, 7)))
    bi8 = jnp.pad(bi, (0, 7))[None, :]
    (wh1, bh1), (wh2, bh2), (wh3, bh3) = params["phi_h"]
    edge_ws = [w1s, w1r, w1g, b1[None, :], w2, b2[None, :],
               wt1, bt1[None, :], wt2, bt2[None, :],
               wx16, bx16, wi8, bi8, m16]
    node_ws = [wh1[:D], wh1[D:], bh1[None, :], wh2, bh2[None, :],
               wh3, bh3[None, :]]
    return edge_ws, node_ws


def kernel(node_positions, node_features, senders, receivers, params):
    n, v, dim = node_positions.shape
    pos_flat = jnp.pad(node_positions.reshape(n, v * dim).astype(jnp.float32),
                       ((0, 0), (0, PT - v * dim))).reshape(n * PT)
    snd = senders.astype(jnp.int32)
    rcv = receivers.astype(jnp.int32)
    edge_ws, node_ws = _prep_weights(params)

    fs, fr = _sc_gather_kernel()(node_features, snd, rcv)
    geo = _sc_geom_kernel()(pos_flat, snd, rcv).reshape(E, PW)
    msg, shf = _edge_mlp(fs, fr, geo, edge_ws)
    zeros = jnp.zeros((N2, D), jnp.float32)
    rcv2 = rcv.reshape(E // IW, IW)
    am = _sc_scatter_kernel(D)(msg, rcv2, zeros)
    ash = _sc_scatter_kernel(PW)(shf, rcv2, jnp.zeros((N2, PW), jnp.float32))
    pos16 = jnp.pad(node_positions.reshape(n, v * dim).astype(jnp.float32),
                    ((0, 0), (0, PW - v * dim)))
    feats_out, pos_out = _node_mlp(am, ash, node_features, pos16, node_ws)
    vectors_out = pos_out[:, :v * dim].reshape(n, v, dim)
    return (vectors_out, feats_out)
